# baseline (speedup 1.0000x reference)
import functools

import jax
import jax.numpy as jnp
from jax import lax
from jax.experimental import pallas as pl
from jax.experimental.pallas import tpu as pltpu
from jax.experimental.pallas import tpu_sc as plsc

_NC = 2
_NS = 16
_NW = _NC * _NS
_CHUNK = 128


@functools.partial(jax.jit, static_argnums=(2,))
def _sc_gather(table, idx3, n_chunks):
    d_model = table.shape[1]
    mesh = plsc.VectorSubcoreMesh(core_axis_name="c", subcore_axis_name="s")

    @functools.partial(
        pl.kernel,
        mesh=mesh,
        out_type=jax.ShapeDtypeStruct((_NW * n_chunks * _CHUNK, d_model),
                                      jnp.float32),
        scratch_types=[
            pltpu.VMEM((n_chunks, _CHUNK), jnp.int32),
            pltpu.VMEM((_CHUNK, d_model), jnp.float32),
            pltpu.SemaphoreType.DMA,
        ],
        compiler_params=pltpu.CompilerParams(use_tc_tiling_on_sc=False),
    )
    def k(table_hbm, idx_hbm, out_hbm, idx_v, rows_v, sem):
        wid = lax.axis_index("s") * _NC + lax.axis_index("c")
        pltpu.sync_copy(idx_hbm.at[wid], idx_v)
        base = wid * (n_chunks * _CHUNK)

        def step(j, carry):
            pltpu.async_copy(table_hbm.at[idx_v.at[j]], rows_v, sem).wait()
            pltpu.sync_copy(rows_v,
                            out_hbm.at[pl.ds(base + j * _CHUNK, _CHUNK)])
            return carry

        lax.fori_loop(0, n_chunks, step, 0)

    return k(table, idx3)


def kernel(x, weights):
    batch, hist = x.shape
    d_model = weights.shape[1]
    total = batch * hist
    n_chunks = total // (_NW * _CHUNK)
    idx3 = x.reshape(_NW, n_chunks, _CHUNK).astype(jnp.int32)
    out = _sc_gather(weights, idx3, n_chunks)
    return out.reshape(batch, hist, d_model)


# speedup vs baseline: 1.1127x; 1.1127x over previous
import functools

import jax
import jax.numpy as jnp
from jax import lax
from jax.experimental import pallas as pl
from jax.experimental.pallas import tpu as pltpu
from jax.experimental.pallas import tpu_sc as plsc

_NC = 2
_NS = 16
_NW = _NC * _NS
_CHUNK = 128
_GK = 4
_NSET = 2
_GKC = _GK * _CHUNK


@functools.partial(jax.jit, static_argnums=(2,))
def _sc_gather(table, idx3, n_chunks):
    d_model = table.shape[1]
    n_groups = n_chunks // _GK
    n_outer = n_groups // _NSET
    per_w = n_chunks * _CHUNK
    mesh = plsc.VectorSubcoreMesh(core_axis_name="c", subcore_axis_name="s")

    @functools.partial(
        pl.kernel,
        mesh=mesh,
        out_type=jax.ShapeDtypeStruct((_NW * per_w, d_model), jnp.float32),
        scratch_types=[
            pltpu.VMEM((n_chunks, _CHUNK), jnp.int32),
            pltpu.VMEM((_NSET, _GKC, d_model), jnp.float32),
            pltpu.SemaphoreType.DMA((_NSET,)),
            pltpu.SemaphoreType.DMA((_NSET,)),
        ],
        compiler_params=pltpu.CompilerParams(use_tc_tiling_on_sc=False),
    )
    def k(table_hbm, idx_hbm, out_hbm, idx_v, rows_v, gsem, ssem):
        wid = lax.axis_index("s") * _NC + lax.axis_index("c")
        pltpu.sync_copy(idx_hbm.at[wid], idx_v)
        base = wid * per_w

        def fire_gathers(g, s):
            for b in range(_GK):
                pltpu.async_copy(
                    table_hbm.at[idx_v.at[g * _GK + b]],
                    rows_v.at[s, pl.ds(b * _CHUNK, _CHUNK)],
                    gsem.at[s])

        def drain_gathers(s):
            for b in range(_GK):
                pltpu.make_async_copy(
                    table_hbm.at[idx_v.at[0]],
                    rows_v.at[s, pl.ds(b * _CHUNK, _CHUNK)],
                    gsem.at[s]).wait()

        def fire_store(g, s):
            pltpu.async_copy(rows_v.at[s],
                             out_hbm.at[pl.ds(base + g * _GKC, _GKC)],
                             ssem.at[s])

        def drain_store(s):
            pltpu.make_async_copy(rows_v.at[s],
                                  out_hbm.at[pl.ds(base, _GKC)],
                                  ssem.at[s]).wait()

        fire_gathers(0, 0)

        def outer(i, carry):
            for p in range(_NSET):
                g = i * _NSET + p
                cur = p
                nxt = (p + 1) % _NSET
                @pl.when(g + 1 - _NSET >= 0)
                def _():
                    drain_store(nxt)

                @pl.when(g + 1 < n_groups)
                def _():
                    fire_gathers(g + 1, nxt)

                drain_gathers(cur)
                fire_store(g, cur)
            return carry

        lax.fori_loop(0, n_outer, outer, 0)
        drain_store((n_groups - 1) % _NSET)

    return k(table, idx3)


def kernel(x, weights):
    batch, hist = x.shape
    d_model = weights.shape[1]
    total = batch * hist
    n_chunks = total // (_NW * _CHUNK)
    idx3 = x.reshape(_NW, n_chunks, _CHUNK).astype(jnp.int32)
    out = _sc_gather(weights, idx3, n_chunks)
    return out.reshape(batch, hist, d_model)
